# baseline (speedup 1.0000x reference)
import functools

import jax
import jax.numpy as jnp
from jax import lax
from jax.experimental import pallas as pl
from jax.experimental.pallas import tpu as pltpu
from jax.experimental.pallas import tpu_sc as plsc

NC = 2
NS = 16
NW = NC * NS
LANES = 16
SP = 256


CH0 = 128


def _emb_body(T, D, V2, rpw, idx_hbm, tab_hbm, pos_hbm, pf_hbm, out_hbm,
              idx0, idx1, jdxa0, jdxb0, jdxa1, jdxb1, pfb0, pfb1, rows_v, pos_v,
              isem0, isem1, gsem0, gsem1, ssem0, ssem1):
  wid = lax.axis_index("s") * NC + lax.axis_index("c")
  base_row = wid * rpw
  npairs = rpw // 2
  idxs = (idx0, idx1)
  jdxas = (jdxa0, jdxa1)
  jdxbs = (jdxb0, jdxb1)
  pfbs = (pfb0, pfb1)
  isems = (isem0, isem1)
  gsems = (gsem0, gsem1)
  ssems = (ssem0, ssem1)

  def idx_copies(b, g):
    return [
        pltpu.make_async_copy(idx_hbm.at[pl.ds((base_row + g) * T, T)],
                              idxs[b].at[pl.ds(0, T)], isems[b]),
        pltpu.make_async_copy(
            pf_hbm.at[pl.ds((base_row + g) * T * LANES, T * LANES)],
            pfbs[b], isems[b]),
    ]

  def gather_copies(b):
    nb2 = jdxbs[b].shape[0]
    return [
        pltpu.make_async_copy(tab_hbm.at[jdxas[b]],
                              rows_v.at[b, pl.ds(0, CH0)], gsems[b]),
        pltpu.make_async_copy(tab_hbm.at[jdxbs[b]],
                              rows_v.at[b, pl.ds(CH0, nb2)], gsems[b]),
    ]

  def store_copy(b, g):
    return pltpu.make_async_copy(rows_v.at[b, pl.ds(0, T)],
                                 out_hbm.at[base_row + g], ssems[b])

  def make_jdx(b):
    def vbody(i, carry):
      iv = jnp.maximum(
          jnp.minimum(idxs[b][pl.ds(i * LANES, LANES)] >> 1, V2 - 1), 0)
      ioff = i * LANES

      @pl.when(i < CH0 // LANES)
      def _a():
        jdxas[b][pl.ds(pl.multiple_of(ioff, LANES), LANES)] = iv

      @pl.when(i >= CH0 // LANES)
      def _b():
        jdxbs[b][pl.ds(pl.multiple_of(ioff - CH0, LANES), LANES)] = iv
      return carry
    lax.fori_loop(0, SP // LANES, vbody, 0, unroll=4)

  def add_pos(b):
    def tbody(t, carry):
      off = pl.multiple_of(t * LANES, LANES)
      pf = pfbs[b][pl.ds(off, LANES)]
      for c in range(0, D, LANES):
        pv = pos_v[t, pl.ds(c, LANES)]
        lo = rows_v[b, t, pl.ds(c, LANES)]
        hi = rows_v[b, t, pl.ds(D + c, LANES)]
        rows_v[b, t, pl.ds(c, LANES)] = lo + pf * (hi - lo) + pv
      return carry
    lax.fori_loop(0, T, tbody, 0, unroll=2)

  def process(g, b):
    nb = 1 - b

    @pl.when(g + 1 < rpw)
    def _fire_next():
      for c in idx_copies(nb, g):
        c.wait()
      make_jdx(nb)

      @pl.when(g >= 1)
      def _reuse():
        store_copy(nb, g).wait()

      for c in gather_copies(nb):
        c.start()

    for c in gather_copies(b):
      c.wait()

    add_pos(b)

    @pl.when(g + 2 < rpw)
    def _prefetch():
      for c in idx_copies(b, g + 2):
        c.start()

    store_copy(b, g).start()

  pltpu.sync_copy(pos_hbm, pos_v)
  for c in idx_copies(0, 0):
    c.start()
  for c in idx_copies(0, 0):
    c.wait()
  make_jdx(0)
  for c in gather_copies(0):
    c.start()
  for c in idx_copies(1, 1):
    c.start()

  def pair(i, carry):
    process(2 * i, 0)
    process(2 * i + 1, 1)
    return carry

  lax.fori_loop(0, npairs, pair, 0)

  store_copy(0, 0).wait()
  store_copy(1, 0).wait()


def kernel(inputs, token_table, pos_table):
  B, T = inputs.shape
  V, D = token_table.shape
  rpw = B // NW

  mesh = plsc.VectorSubcoreMesh(core_axis_name="c", subcore_axis_name="s",
                                num_cores=NC, num_subcores=NS)
  emb = pl.kernel(
      functools.partial(_emb_body, T, D, V // 2, rpw),
      out_type=jax.ShapeDtypeStruct((B, T, 2 * D), jnp.float32),
      mesh=mesh,
      compiler_params=pltpu.CompilerParams(use_tc_tiling_on_sc=True),
      scratch_types=[
          pltpu.VMEM((SP,), jnp.int32),
          pltpu.VMEM((SP,), jnp.int32),
          pltpu.VMEM((128,), jnp.int32),
          pltpu.VMEM((SP - 128,), jnp.int32),
          pltpu.VMEM((128,), jnp.int32),
          pltpu.VMEM((SP - 128,), jnp.int32),
          pltpu.VMEM((T * LANES,), jnp.float32),
          pltpu.VMEM((T * LANES,), jnp.float32),
          pltpu.VMEM((2, 128 + SP - 128, 2 * D), jnp.float32),
          pltpu.VMEM((T, D), jnp.float32),
          pltpu.SemaphoreType.DMA,
          pltpu.SemaphoreType.DMA,
          pltpu.SemaphoreType.DMA,
          pltpu.SemaphoreType.DMA,
          pltpu.SemaphoreType.DMA,
          pltpu.SemaphoreType.DMA,
      ],
  )
  tab2 = token_table.reshape(V // 2, 2 * D)
  idx_flat = inputs.reshape(-1).astype(jnp.int32)
  pf_exp = jnp.broadcast_to(
      (idx_flat & 1).astype(jnp.float32)[:, None], (B * T, LANES)).reshape(-1)
  out = emb(idx_flat, tab2, pos_table, pf_exp)
  return out[:, :, :D]


# speedup vs baseline: 5.2638x; 5.2638x over previous
import functools

import jax
import jax.numpy as jnp
from jax import lax
from jax.experimental import pallas as pl
from jax.experimental.pallas import tpu as pltpu
from jax.experimental.pallas import tpu_sc as plsc

NC = 2
NS = 16
NW = NC * NS
LANES = 16
K = 4


def _row_chunks(T):
  half = (T // 2 + 7) // 8 * 8
  return ((0, half), (half, T - half))


def _emb_body(T, D, rpw, idx_hbm, tab_hbm, pos_hbm, out_hbm,
              idx_v, rows_v, pos_v,
              isem0, isem1, gsem0, gsem1, ssem0, ssem1):
  wid = lax.axis_index("s") * NC + lax.axis_index("c")
  base_row = wid * rpw
  ngroups = rpw // K
  npairs = ngroups // 2
  isems = (isem0, isem1)
  gsems = (gsem0, gsem1)
  ssems = (ssem0, ssem1)

  def idx_copy(b, g):
    return pltpu.make_async_copy(idx_hbm.at[pl.ds(base_row + g * K, K)],
                                 idx_v.at[b], isems[b])

  def gather_copies(b):
    cps = []
    for k in range(K):
      for off, sz in _row_chunks(T):
        cps.append(pltpu.make_async_copy(
            tab_hbm.at[idx_v.at[b, k, pl.ds(off, sz)]],
            rows_v.at[b, k, pl.ds(off, sz)], gsems[b]))
    return cps

  def store_copy(b, g):
    return pltpu.make_async_copy(rows_v.at[b],
                                 out_hbm.at[pl.ds(base_row + g * K, K)],
                                 ssems[b])

  def add_pos(b):
    def tbody(t, carry):
      for c in range(0, D, LANES):
        pv = pos_v[t, pl.ds(c, LANES)]
        for k in range(K):
          rows_v[b, k, t, pl.ds(c, LANES)] = (
              rows_v[b, k, t, pl.ds(c, LANES)] + pv)
      return carry
    lax.fori_loop(0, T, tbody, 0, unroll=2)

  def process(g, b, first, fire_next, prefetch_idx):
    nb = 1 - b
    if fire_next:
      idx_copy(nb, g).wait()
      if not first:
        store_copy(nb, g).wait()
      for c in gather_copies(nb):
        c.start()
    for c in gather_copies(b):
      c.wait()
    if prefetch_idx:
      idx_copy(b, g + 2).start()
    add_pos(b)
    store_copy(b, g).start()

  pltpu.sync_copy(pos_hbm, pos_v)
  idx_copy(0, 0).start()
  idx_copy(0, 0).wait()
  for c in gather_copies(0):
    c.start()
  idx_copy(1, 1).start()

  def pair(i, first_pair, last_pair):
    g = 2 * i
    process(g, 0, first=first_pair, fire_next=True,
            prefetch_idx=not last_pair)
    process(g + 1, 1, first=False, fire_next=not last_pair,
            prefetch_idx=not last_pair)

  pair(0, True, False)
  lax.fori_loop(1, npairs - 1, lambda i, c: (pair(i, False, False), c)[1], 0)
  pair(npairs - 1, False, True)

  store_copy(0, 0).wait()
  store_copy(1, 0).wait()


def kernel(inputs, token_table, pos_table):
  B, T = inputs.shape
  V, D = token_table.shape
  rpw = B // NW

  mesh = plsc.VectorSubcoreMesh(core_axis_name="c", subcore_axis_name="s",
                                num_cores=NC, num_subcores=NS)
  emb = pl.kernel(
      functools.partial(_emb_body, T, D, rpw),
      out_type=jax.ShapeDtypeStruct((B, T, D), jnp.float32),
      mesh=mesh,
      compiler_params=pltpu.CompilerParams(use_tc_tiling_on_sc=False),
      scratch_types=[
          pltpu.VMEM((2, K, T), jnp.int32),
          pltpu.VMEM((2, K, T, D), jnp.float32),
          pltpu.VMEM((T, D), jnp.float32),
          pltpu.SemaphoreType.DMA,
          pltpu.SemaphoreType.DMA,
          pltpu.SemaphoreType.DMA,
          pltpu.SemaphoreType.DMA,
          pltpu.SemaphoreType.DMA,
          pltpu.SemaphoreType.DMA,
      ],
  )
  return emb(inputs.astype(jnp.int32), token_table, pos_table)
